# baseline (speedup 1.0000x reference)
import jax
import jax.numpy as jnp
from jax import lax
from jax.experimental import pallas as pl
from jax.experimental.pallas import tpu as pltpu
from jax.experimental.pallas import tpu_sc as plsc

K = 32
B = 16384

NC = 2
NS = 16
NW = NC * NS
BPW = B // NW
NCHUNK = BPW // 128
NGROUP = BPW // 16


def _gmf_body(x_hbm, wb_hbm, user_hbm, item_hbm, out_hbm,
              xv, uidx, iidx, ucols, icols, wbv, outv, sem):
    wid = lax.axis_index("s") * NC + lax.axis_index("c")
    base = wid * BPW

    pltpu.sync_copy(x_hbm.at[pl.ds(base * 2, 2 * BPW)], xv)
    pltpu.sync_copy(wb_hbm, wbv)

    iota16 = lax.iota(jnp.int32, 16)

    for j in range(NGROUP):
        chunk, off = (j * 16) // 128, (j * 16) % 128
        uidx[chunk, pl.ds(off, 16)] = plsc.load_gather(xv, [iota16 * 2 + (j * 32)])
        iidx[chunk, pl.ds(off, 16)] = plsc.load_gather(xv, [iota16 * 2 + (j * 32 + 1)])

    copies = []
    for k in range(K):
        for c in range(NCHUNK):
            copies.append(pltpu.async_copy(
                user_hbm.at[k].at[uidx.at[c]],
                ucols.at[k, pl.ds(c * 128, 128)], sem))
            copies.append(pltpu.async_copy(
                item_hbm.at[k].at[iidx.at[c]],
                icols.at[k, pl.ds(c * 128, 128)], sem))
    for cp in copies:
        cp.wait()

    w0 = wbv[pl.ds(0, 16)]
    w1 = wbv[pl.ds(16, 16)]
    bias = wbv[pl.ds(24, 16)][8]

    def group(g, carry):
        acc = jnp.zeros((16,), jnp.float32)
        for k in range(K):
            ucol = ucols[k, pl.ds(g * 16, 16)]
            icol = icols[k, pl.ds(g * 16, 16)]
            wk = w0[k] if k < 16 else w1[k - 16]
            acc = acc + ucol * icol * wk
        z = acc + bias
        outv[pl.ds(g * 16, 16)] = 1.0 / (1.0 + jnp.exp(-z))
        return carry

    lax.fori_loop(0, NGROUP, group, None)

    pltpu.sync_copy(outv, out_hbm.at[pl.ds(base, BPW)])


@jax.jit
def kernel(x, user_table, item_table, W, b):
    wb = jnp.concatenate([W.reshape(K), jnp.pad(b, (0, 7))]).astype(jnp.float32)
    mesh = plsc.VectorSubcoreMesh(core_axis_name="c", subcore_axis_name="s")
    out = pl.kernel(
        _gmf_body,
        out_type=jax.ShapeDtypeStruct((B,), jnp.float32),
        mesh=mesh,
        compiler_params=pltpu.CompilerParams(
            needs_layout_passes=False, use_tc_tiling_on_sc=False),
        scratch_types=[
            pltpu.VMEM((2 * BPW,), jnp.int32),
            pltpu.VMEM((NCHUNK, 128), jnp.int32),
            pltpu.VMEM((NCHUNK, 128), jnp.int32),
            pltpu.VMEM((K, BPW), jnp.float32),
            pltpu.VMEM((K, BPW), jnp.float32),
            pltpu.VMEM((40,), jnp.float32),
            pltpu.VMEM((BPW,), jnp.float32),
            pltpu.SemaphoreType.DMA,
        ],
    )(x.astype(jnp.int32).reshape(2 * B),
      wb,
      user_table.T,
      item_table.T)
    return out.reshape(B, 1, 1)


# speedup vs baseline: 2.0888x; 2.0888x over previous
import functools

import jax
import jax.numpy as jnp
from jax import lax
from jax.experimental import pallas as pl
from jax.experimental.pallas import tpu as pltpu
from jax.experimental.pallas import tpu_sc as plsc

K = 32
B = 16384
M = 1000000

NC = 2
NS = 16
NW = NC * NS
BPW = B // NW
NCHUNK = BPW // 128
NGROUP = BPW // 16

TCOLS = 8192
TGRID = (M + TCOLS - 1) // TCOLS


def _transpose_body(in_ref, out_ref):
    v = in_ref[...].reshape(2, 16, TCOLS)
    out_ref[...] = v.swapaxes(1, 2).swapaxes(0, 1).reshape(2 * TCOLS, 16)


def _to_half_rows(table_t):
    return pl.pallas_call(
        _transpose_body,
        grid=(TGRID,),
        in_specs=[pl.BlockSpec((K, TCOLS), lambda i: (0, i))],
        out_specs=pl.BlockSpec((2 * TCOLS, 16), lambda i: (i, 0)),
        out_shape=jax.ShapeDtypeStruct((2 * M, 16), jnp.float32),
    )(table_t)


def _gmf_body(x_hbm, wb_hbm, user_hbm, item_hbm, out_hbm,
              xv, uidx_lo, uidx_hi, iidx_lo, iidx_hi,
              ulo, uhi, ilo, ihi, wbv, outv, sem):
    wid = lax.axis_index("s") * NC + lax.axis_index("c")
    base = wid * BPW

    pltpu.sync_copy(x_hbm.at[pl.ds(base * 2, 2 * BPW)], xv)
    pltpu.sync_copy(wb_hbm, wbv)

    iota16 = lax.iota(jnp.int32, 16)

    for j in range(NGROUP):
        chunk, off = (j * 16) // 128, (j * 16) % 128
        ucol = plsc.load_gather(xv, [iota16 * 2 + (j * 32)])
        icol = plsc.load_gather(xv, [iota16 * 2 + (j * 32 + 1)])
        uidx_lo[chunk, pl.ds(off, 16)] = ucol * 2
        uidx_hi[chunk, pl.ds(off, 16)] = ucol * 2 + 1
        iidx_lo[chunk, pl.ds(off, 16)] = icol * 2
        iidx_hi[chunk, pl.ds(off, 16)] = icol * 2 + 1

    copies = []
    for c in range(NCHUNK):
        for idxref, table, dst in ((uidx_lo, user_hbm, ulo),
                                   (uidx_hi, user_hbm, uhi),
                                   (iidx_lo, item_hbm, ilo),
                                   (iidx_hi, item_hbm, ihi)):
            copies.append(pltpu.async_copy(
                table.at[idxref.at[c]], dst.at[pl.ds(c * 128, 128)], sem))
    for cp in copies:
        cp.wait()

    w0 = wbv[pl.ds(0, 16)]
    w1 = wbv[pl.ds(16, 16)]
    bias = wbv[pl.ds(24, 16)][8]

    def group(g, carry):
        rows = iota16 + g * 16
        acc = jnp.zeros((16,), jnp.float32)
        for k in range(K):
            uref = ulo if k < 16 else uhi
            iref = ilo if k < 16 else ihi
            kvec = jnp.full((16,), k % 16, jnp.int32)
            ucol = plsc.load_gather(uref, [rows, kvec])
            icol = plsc.load_gather(iref, [rows, kvec])
            wk = w0[k] if k < 16 else w1[k - 16]
            acc = acc + ucol * icol * wk
        z = acc + bias
        outv[pl.ds(g * 16, 16)] = 1.0 / (1.0 + jnp.exp(-z))
        return carry

    lax.fori_loop(0, NGROUP, group, None)

    pltpu.sync_copy(outv, out_hbm.at[pl.ds(base, BPW)])


@jax.jit
def kernel(x, user_table, item_table, W, b):
    wb = jnp.concatenate([W.reshape(K), jnp.pad(b, (0, 7))]).astype(jnp.float32)
    user_rm = _to_half_rows(user_table.T)
    item_rm = _to_half_rows(item_table.T)
    mesh = plsc.VectorSubcoreMesh(core_axis_name="c", subcore_axis_name="s")
    out = pl.kernel(
        _gmf_body,
        out_type=jax.ShapeDtypeStruct((B,), jnp.float32),
        mesh=mesh,
        compiler_params=pltpu.CompilerParams(
            needs_layout_passes=False, use_tc_tiling_on_sc=False),
        scratch_types=[
            pltpu.VMEM((2 * BPW,), jnp.int32),
            pltpu.VMEM((NCHUNK, 128), jnp.int32),
            pltpu.VMEM((NCHUNK, 128), jnp.int32),
            pltpu.VMEM((NCHUNK, 128), jnp.int32),
            pltpu.VMEM((NCHUNK, 128), jnp.int32),
            pltpu.VMEM((BPW, 16), jnp.float32),
            pltpu.VMEM((BPW, 16), jnp.float32),
            pltpu.VMEM((BPW, 16), jnp.float32),
            pltpu.VMEM((BPW, 16), jnp.float32),
            pltpu.VMEM((40,), jnp.float32),
            pltpu.VMEM((BPW,), jnp.float32),
            pltpu.SemaphoreType.DMA,
        ],
    )(x.astype(jnp.int32).reshape(2 * B), wb, user_rm, item_rm)
    return out.reshape(B, 1, 1)


# speedup vs baseline: 4.3531x; 2.0840x over previous
import functools

import jax
import jax.numpy as jnp
from jax import lax
from jax.experimental import pallas as pl
from jax.experimental.pallas import tpu as pltpu
from jax.experimental.pallas import tpu_sc as plsc

K = 32
B = 16384
M = 1000000

NC = 2
NS = 16
NW = NC * NS
BPW = B // NW
NCHUNK = BPW // 128
NGROUP = BPW // 16

TCOLS = 8192
TGRID = (M + TCOLS - 1) // TCOLS


def _transpose_body(in_ref, out_ref):
    out_ref[...] = in_ref[...].T


def _to_half_rows(table_t):
    rm = pl.pallas_call(
        _transpose_body,
        grid=(TGRID,),
        in_specs=[pl.BlockSpec((K, TCOLS), lambda i: (0, i))],
        out_specs=pl.BlockSpec((TCOLS, K), lambda i: (i, 0)),
        out_shape=jax.ShapeDtypeStruct((M, K), jnp.float32),
    )(table_t)
    return rm.reshape(2 * M, 16)


def _gmf_body(x_hbm, wb_hbm, user_hbm, item_hbm, out_hbm,
              xv, uidx_lo, uidx_hi, iidx_lo, iidx_hi,
              ulo, uhi, ilo, ihi, wbv, outv, sem):
    wid = lax.axis_index("s") * NC + lax.axis_index("c")
    base = wid * BPW

    pltpu.sync_copy(x_hbm.at[pl.ds(base * 2, 2 * BPW)], xv)
    pltpu.sync_copy(wb_hbm, wbv)

    iota16 = lax.iota(jnp.int32, 16)

    for j in range(NGROUP):
        chunk, off = (j * 16) // 128, (j * 16) % 128
        ucol = plsc.load_gather(xv, [iota16 * 2 + (j * 32)])
        icol = plsc.load_gather(xv, [iota16 * 2 + (j * 32 + 1)])
        uidx_lo[chunk, pl.ds(off, 16)] = ucol * 2
        uidx_hi[chunk, pl.ds(off, 16)] = ucol * 2 + 1
        iidx_lo[chunk, pl.ds(off, 16)] = icol * 2
        iidx_hi[chunk, pl.ds(off, 16)] = icol * 2 + 1

    copies = []
    for c in range(NCHUNK):
        for idxref, table, dst in ((uidx_lo, user_hbm, ulo),
                                   (uidx_hi, user_hbm, uhi),
                                   (iidx_lo, item_hbm, ilo),
                                   (iidx_hi, item_hbm, ihi)):
            copies.append(pltpu.async_copy(
                table.at[idxref.at[c]], dst.at[pl.ds(c * 128, 128)], sem))
    for cp in copies:
        cp.wait()

    w0 = wbv[pl.ds(0, 16)]
    w1 = wbv[pl.ds(16, 16)]
    bias = wbv[pl.ds(24, 16)][8]

    def group(g, carry):
        rows = iota16 + g * 16
        acc = jnp.zeros((16,), jnp.float32)
        for k in range(K):
            uref = ulo if k < 16 else uhi
            iref = ilo if k < 16 else ihi
            kvec = jnp.full((16,), k % 16, jnp.int32)
            ucol = plsc.load_gather(uref, [rows, kvec])
            icol = plsc.load_gather(iref, [rows, kvec])
            wk = w0[k] if k < 16 else w1[k - 16]
            acc = acc + ucol * icol * wk
        z = acc + bias
        outv[pl.ds(g * 16, 16)] = 1.0 / (1.0 + jnp.exp(-z))
        return carry

    lax.fori_loop(0, NGROUP, group, None)

    pltpu.sync_copy(outv, out_hbm.at[pl.ds(base, BPW)])


@jax.jit
def kernel(x, user_table, item_table, W, b):
    wb = jnp.concatenate([W.reshape(K), jnp.pad(b, (0, 7))]).astype(jnp.float32)
    user_rm = _to_half_rows(user_table.T)
    item_rm = _to_half_rows(item_table.T)
    mesh = plsc.VectorSubcoreMesh(core_axis_name="c", subcore_axis_name="s")
    out = pl.kernel(
        _gmf_body,
        out_type=jax.ShapeDtypeStruct((B,), jnp.float32),
        mesh=mesh,
        compiler_params=pltpu.CompilerParams(
            needs_layout_passes=False, use_tc_tiling_on_sc=False),
        scratch_types=[
            pltpu.VMEM((2 * BPW,), jnp.int32),
            pltpu.VMEM((NCHUNK, 128), jnp.int32),
            pltpu.VMEM((NCHUNK, 128), jnp.int32),
            pltpu.VMEM((NCHUNK, 128), jnp.int32),
            pltpu.VMEM((NCHUNK, 128), jnp.int32),
            pltpu.VMEM((BPW, 16), jnp.float32),
            pltpu.VMEM((BPW, 16), jnp.float32),
            pltpu.VMEM((BPW, 16), jnp.float32),
            pltpu.VMEM((BPW, 16), jnp.float32),
            pltpu.VMEM((40,), jnp.float32),
            pltpu.VMEM((BPW,), jnp.float32),
            pltpu.SemaphoreType.DMA,
        ],
    )(x.astype(jnp.int32).reshape(2 * B), wb, user_rm, item_rm)
    return out.reshape(B, 1, 1)


# speedup vs baseline: 4.8144x; 1.1060x over previous
import jax
import jax.numpy as jnp
from jax import lax
from jax.experimental import pallas as pl
from jax.experimental.pallas import tpu as pltpu
from jax.experimental.pallas import tpu_sc as plsc

K = 32
B = 16384

NC = 2
NS = 16
NW = NC * NS
BPW = B // NW
NCHUNK = BPW // 128
NGROUP = BPW // 16


def _gmf_body(x_hbm, wb_hbm, user_hbm, item_hbm, out_hbm,
              xv, uidx, iidx, urows, irows, wbv, outv, sem):
    wid = lax.axis_index("s") * NC + lax.axis_index("c")
    base = wid * BPW

    pltpu.sync_copy(x_hbm.at[pl.ds(base * 2, 2 * BPW)], xv)
    pltpu.sync_copy(wb_hbm, wbv)

    iota16 = lax.iota(jnp.int32, 16)

    for j in range(NGROUP):
        chunk, off = (j * 16) // 128, (j * 16) % 128
        uidx[chunk, pl.ds(off, 16)] = plsc.load_gather(xv, [iota16 * 2 + (j * 32)])
        iidx[chunk, pl.ds(off, 16)] = plsc.load_gather(xv, [iota16 * 2 + (j * 32 + 1)])

    copies = []
    for c in range(NCHUNK):
        copies.append(pltpu.async_copy(
            user_hbm.at[uidx.at[c]], urows.at[pl.ds(c * 128, 128)], sem))
        copies.append(pltpu.async_copy(
            item_hbm.at[iidx.at[c]], irows.at[pl.ds(c * 128, 128)], sem))
    for cp in copies:
        cp.wait()

    we = wbv[pl.ds(0, 16)]
    wo = wbv[pl.ds(16, 16)]
    bias = wbv[pl.ds(24, 16)][8]
    lane0 = iota16 == 0

    def row(r, carry):
        u = urows[r, :]
        i = irows[r, :]
        ue, uo = plsc.unpack(u, format=plsc.PackFormat.INTERLEAVED)
        ie, io = plsc.unpack(i, format=plsc.PackFormat.INTERLEAVED)
        t = ue * ie * we + uo * io * wo
        z = plsc.cumsum(t)[15] + bias
        plsc.store_scatter(outv, [jnp.full((16,), r, jnp.int32)],
                           jnp.full((16,), z), mask=lane0)
        return carry

    lax.fori_loop(0, BPW, row, None)

    def sig(g, carry):
        z = outv[pl.ds(g * 16, 16)]
        outv[pl.ds(g * 16, 16)] = 1.0 / (1.0 + jnp.exp(-z))
        return carry

    lax.fori_loop(0, NGROUP, sig, None)

    pltpu.sync_copy(outv, out_hbm.at[pl.ds(base, BPW)])


@jax.jit
def kernel(x, user_table, item_table, W, b):
    wf = W.reshape(K)
    wb = jnp.concatenate([wf[0::2], wf[1::2], jnp.pad(b, (0, 7))]).astype(jnp.float32)
    mesh = plsc.VectorSubcoreMesh(core_axis_name="c", subcore_axis_name="s")
    out = pl.kernel(
        _gmf_body,
        out_type=jax.ShapeDtypeStruct((B,), jnp.float32),
        mesh=mesh,
        compiler_params=pltpu.CompilerParams(
            needs_layout_passes=False, use_tc_tiling_on_sc=False),
        scratch_types=[
            pltpu.VMEM((2 * BPW,), jnp.int32),
            pltpu.VMEM((NCHUNK, 128), jnp.int32),
            pltpu.VMEM((NCHUNK, 128), jnp.int32),
            pltpu.VMEM((BPW, K), jnp.bfloat16),
            pltpu.VMEM((BPW, K), jnp.bfloat16),
            pltpu.VMEM((40,), jnp.float32),
            pltpu.VMEM((BPW,), jnp.float32),
            pltpu.SemaphoreType.DMA,
        ],
    )(x.astype(jnp.int32).reshape(2 * B),
      wb,
      user_table.astype(jnp.bfloat16),
      item_table.astype(jnp.bfloat16))
    return out.reshape(B, 1, 1)
